# Initial kernel scaffold; baseline (speedup 1.0000x reference)
import jax
import jax.numpy as jnp
from jax.experimental import pallas as pl


def kernel(features, edges, W1, att_src1, att_dst1, b1, W2, att_src2, att_dst2, b2):
    raise NotImplementedError("write your pallas kernel here")


# speedup vs baseline: 12.9515x; 12.9515x over previous
import dataclasses
import functools

import jax
import jax.numpy as jnp
from jax import lax
from jax.experimental import pallas as pl
from jax.experimental.pallas import tpu as pltpu
from jax.experimental.pallas import tpu_sc as plsc

_N = 10000
_E = 320000
_F = 128
_H = 128
_C = 16

_D1 = 80
_D2 = 16

_NCORES = 2
_NSUB = 16
_CH = _E // _NSUB
_B = 80
_NB = _CH // _B
_NPAD = 10240
_RPS = _NPAD // _NSUB
_ZR = 128


def _dense1(features, w1, att_s, att_d):
  def body(x_ref, w_ref, s_ref, d_ref, tab_ref, as_ref, ad_ref, m_ref):
    xw = jnp.dot(x_ref[...], w_ref[...], preferred_element_type=jnp.float32)
    hh = _H // 2
    ones_col = (lax.broadcasted_iota(jnp.int32, (_N, _D1 - hh), 1) == 0)
    ones_col = ones_col.astype(jnp.float32)
    tab_ref[0, :, :hh] = xw[:, :hh]
    tab_ref[0, :, hh:] = ones_col
    tab_ref[1, :, :hh] = xw[:, hh:]
    tab_ref[1, :, hh:] = ones_col
    a_s = jnp.sum(xw * s_ref[...], axis=1)
    a_d = jnp.sum(xw * d_ref[...], axis=1)
    as_ref[...] = a_s[None, :]
    ad_ref[...] = a_d[None, :]
    m_ref[...] = jnp.concatenate(
        [jnp.max(a_s)[None, None], jnp.max(a_d)[None, None]], axis=1)

  return pl.pallas_call(
      body,
      out_shape=[
          jax.ShapeDtypeStruct((_NCORES, _N, _D1), jnp.float32),
          jax.ShapeDtypeStruct((1, _N), jnp.float32),
          jax.ShapeDtypeStruct((1, _N), jnp.float32),
          jax.ShapeDtypeStruct((1, 2), jnp.float32),
      ],
  )(features, w1, att_s, att_d)


def _mid(partial1, b1, w2, att_s, att_d):
  def body(p_ref, b_ref, w_ref, s_ref, d_ref, tab_ref, as_ref, ad_ref, m_ref):
    hh = _H // 2
    num = jnp.concatenate(
        [p_ref[0, :_N, :hh], p_ref[1, :_N, :hh]], axis=1)
    den = p_ref[0, :_N, hh:hh + 1] + 1e-16
    h = jax.nn.relu(num / den + b_ref[...])
    xw = jnp.dot(h, w_ref[...], preferred_element_type=jnp.float32)
    ch = _C // 2
    ones_col = (lax.broadcasted_iota(jnp.int32, (_N, _D2 - ch), 1) == 0)
    ones_col = ones_col.astype(jnp.float32)
    tab_ref[0, :, :ch] = xw[:, :ch]
    tab_ref[0, :, ch:] = ones_col
    tab_ref[1, :, :ch] = xw[:, ch:]
    tab_ref[1, :, ch:] = ones_col
    a_s = jnp.sum(xw * s_ref[...], axis=1)
    a_d = jnp.sum(xw * d_ref[...], axis=1)
    as_ref[...] = a_s[None, :]
    ad_ref[...] = a_d[None, :]
    m_ref[...] = jnp.concatenate(
        [jnp.max(a_s)[None, None], jnp.max(a_d)[None, None]], axis=1)

  return pl.pallas_call(
      body,
      out_shape=[
          jax.ShapeDtypeStruct((_NCORES, _N, _D2), jnp.float32),
          jax.ShapeDtypeStruct((1, _N), jnp.float32),
          jax.ShapeDtypeStruct((1, _N), jnp.float32),
          jax.ShapeDtypeStruct((1, 2), jnp.float32),
      ],
  )(partial1, b1, w2, att_s, att_d)


def _final(partial2, b2):
  def body(p_ref, b_ref, o_ref):
    ch = _C // 2
    num = jnp.concatenate(
        [p_ref[0, :_N, :ch], p_ref[1, :_N, :ch]], axis=1)
    den = p_ref[0, :_N, ch:ch + 1] + 1e-16
    o = num / den + b_ref[...]
    mx = jnp.max(o, axis=1, keepdims=True)
    shifted = o - mx
    lse = jnp.log(jnp.sum(jnp.exp(shifted), axis=1, keepdims=True))
    o_ref[...] = shifted - lse

  return pl.pallas_call(
      body,
      out_shape=jax.ShapeDtypeStruct((_N, _C), jnp.float32),
  )(partial2, b2)


def _sc_layer(tabs, src, dst, a_s, a_d, m16, d):
  nd = d // 16
  mesh = plsc.VectorSubcoreMesh(core_axis_name="c", subcore_axis_name="s")
  cp = pltpu.CompilerParams()
  if "needs_layout_passes" in pltpu.CompilerParams.__dataclass_fields__:
    cp = dataclasses.replace(cp, needs_layout_passes=False)
  if "use_tc_tiling_on_sc" in pltpu.CompilerParams.__dataclass_fields__:
    cp = dataclasses.replace(cp, use_tc_tiling_on_sc=False)

  @functools.partial(
      pl.kernel,
      mesh=mesh,
      compiler_params=cp,
      out_type=jax.ShapeDtypeStruct((_NCORES, _NPAD, d), jnp.float32),
      scratch_types=[
          pltpu.VMEM((_N,), jnp.float32),
          pltpu.VMEM((_N,), jnp.float32),
          pltpu.VMEM((16,), jnp.float32),
          pltpu.VMEM((_B,), jnp.int32),
          pltpu.VMEM((_B,), jnp.int32),
          pltpu.VMEM((1, _B), jnp.int32),
          pltpu.VMEM((_B,), jnp.float32),
          pltpu.VMEM((_B, d), jnp.float32),
          pltpu.VMEM((_ZR, d), jnp.float32),
          pltpu.VMEM_SHARED((_NPAD, d), jnp.float32),
          pltpu.SemaphoreType.DMA,
      ],
  )
  def k(tab_hbm, src_hbm, dst_hbm, as_hbm, ad_hbm, m_hbm, out_hbm,
        as_v, ad_v, m_v, si_v, di_v, ds_v, w_v, rows_v, z_v, acc_sh, sem):
    c = lax.axis_index("c")
    s = lax.axis_index("s")

    pltpu.sync_copy(as_hbm, as_v)
    pltpu.sync_copy(ad_hbm, ad_v)
    pltpu.sync_copy(m_hbm, m_v)

    @pl.loop(0, _ZR)
    def _(r):
      for ch in range(nd):
        z_v[r, pl.ds(ch * 16, 16)] = jnp.zeros((16,), jnp.float32)

    @pl.loop(0, _RPS // _ZR)
    def _(t):
      pltpu.sync_copy(z_v, acc_sh.at[pl.ds(s * _RPS + t * _ZR, _ZR)])

    plsc.subcore_barrier()

    mvec = m_v[...]
    base0 = s * _CH
    my_tab = tab_hbm.at[c]

    @pl.loop(0, _NB)
    def _(b):
      base = base0 + b * _B
      pltpu.sync_copy(src_hbm.at[pl.ds(base, _B)], si_v)
      pltpu.sync_copy(dst_hbm.at[pl.ds(base, _B)], di_v)
      pltpu.sync_copy(dst_hbm.at[pl.ds(base, _B)], ds_v.at[0])
      gat = pltpu.async_copy(my_tab.at[si_v], rows_v, sem)

      @pl.loop(0, _B, step=16)
      def _(j):
        sidx = si_v[pl.ds(j, 16)]
        didx = di_v[pl.ds(j, 16)]
        e = plsc.load_gather(as_v, [sidx]) + plsc.load_gather(ad_v, [didx])
        e = jnp.maximum(e, 0.2 * e)
        w_v[pl.ds(j, 16)] = jnp.exp(e - mvec)

      gat.wait()

      @pl.loop(0, _B)
      def _(i):
        ws = plsc.load_gather(w_v, [jnp.full((16,), 0, jnp.int32) + i])
        for ch in range(nd):
          sl = pl.ds(ch * 16, 16)
          rows_v[i, sl] = rows_v[i, sl] * ws

      pltpu.sync_copy(rows_v, acc_sh.at[ds_v.at[0]], add=True)

    plsc.subcore_barrier()

    @pl.loop(0, _RPS // _ZR)
    def _(t):
      r0 = s * _RPS + t * _ZR
      pltpu.sync_copy(acc_sh.at[pl.ds(r0, _ZR)], out_hbm.at[c, pl.ds(r0, _ZR)])

  return k(tabs, src, dst, a_s, a_d, m16)


def kernel(features, edges, W1, att_src1, att_dst1, b1,
           W2, att_src2, att_dst2, b2):
  src = edges[0]
  dst = edges[1]

  tabs1, a1s, a1d, m1 = _dense1(features, W1, att_src1, att_dst1)
  big_m1 = jnp.maximum(m1[0, 0] + m1[0, 1], 0.0)
  m16_1 = jnp.full((16,), big_m1, jnp.float32)
  part1 = _sc_layer(tabs1, src, dst, a1s.reshape(-1), a1d.reshape(-1),
                    m16_1, _D1)

  tabs2, a2s, a2d, m2 = _mid(part1, b1.reshape(1, _H), W2, att_src2, att_dst2)
  big_m2 = jnp.maximum(m2[0, 0] + m2[0, 1], 0.0)
  m16_2 = jnp.full((16,), big_m2, jnp.float32)
  part2 = _sc_layer(tabs2, src, dst, a2s.reshape(-1), a2d.reshape(-1),
                    m16_2, _D2)

  return _final(part2, b2.reshape(1, _C))


# speedup vs baseline: 27.6420x; 2.1343x over previous
import dataclasses
import functools

import jax
import jax.numpy as jnp
from jax import lax
from jax.experimental import pallas as pl
from jax.experimental.pallas import tpu as pltpu
from jax.experimental.pallas import tpu_sc as plsc

_N = 10000
_E = 320000
_F = 128
_H = 128
_C = 16

_D1 = 80
_D2 = 16

_NCORES = 2
_NSUB = 16
_CH = _E // _NSUB
_B = 80
_NB = _CH // _B
_NCH = _N // _B
_DT = 4000
_SEG = _CH // _DT


def _dense1(features, w1, att_s, att_d):
  def body(x_ref, w_ref, s_ref, d_ref, tab_ref, as_ref, ad_ref, m_ref):
    xw = jnp.dot(x_ref[...], w_ref[...], preferred_element_type=jnp.float32)
    hh = _H // 2
    ones_col = (lax.broadcasted_iota(jnp.int32, (_N, _D1 - hh), 1) == 0)
    ones_col = ones_col.astype(jnp.float32)
    tab_ref[0, :, :hh] = xw[:, :hh]
    tab_ref[0, :, hh:] = ones_col
    tab_ref[1, :, :hh] = xw[:, hh:]
    tab_ref[1, :, hh:] = ones_col
    a_s = jnp.sum(xw * s_ref[...], axis=1)
    a_d = jnp.sum(xw * d_ref[...], axis=1)
    as_ref[...] = a_s[None, :]
    ad_ref[...] = a_d[None, :]
    m_ref[...] = jnp.concatenate(
        [jnp.max(a_s)[None, None], jnp.max(a_d)[None, None]], axis=1)

  return pl.pallas_call(
      body,
      out_shape=[
          jax.ShapeDtypeStruct((_NCORES, _N, _D1), jnp.float32),
          jax.ShapeDtypeStruct((1, _N), jnp.float32),
          jax.ShapeDtypeStruct((1, _N), jnp.float32),
          jax.ShapeDtypeStruct((1, 2), jnp.float32),
      ],
  )(features, w1, att_s, att_d)


def _mid(partial1, b1, w2, att_s, att_d):
  def body(p_ref, b_ref, w_ref, s_ref, d_ref, tab_ref, as_ref, ad_ref, m_ref):
    hh = _H // 2
    num = jnp.concatenate(
        [p_ref[0, :_N, :hh], p_ref[1, :_N, :hh]], axis=1)
    den = p_ref[0, :_N, hh:hh + 1] + 1e-16
    h = jax.nn.relu(num / den + b_ref[...])
    xw = jnp.dot(h, w_ref[...], preferred_element_type=jnp.float32)
    ch = _C // 2
    ones_col = (lax.broadcasted_iota(jnp.int32, (_N, _D2 - ch), 1) == 0)
    ones_col = ones_col.astype(jnp.float32)
    tab_ref[0, :, :ch] = xw[:, :ch]
    tab_ref[0, :, ch:] = ones_col
    tab_ref[1, :, :ch] = xw[:, ch:]
    tab_ref[1, :, ch:] = ones_col
    a_s = jnp.sum(xw * s_ref[...], axis=1)
    a_d = jnp.sum(xw * d_ref[...], axis=1)
    as_ref[...] = a_s[None, :]
    ad_ref[...] = a_d[None, :]
    m_ref[...] = jnp.concatenate(
        [jnp.max(a_s)[None, None], jnp.max(a_d)[None, None]], axis=1)

  return pl.pallas_call(
      body,
      out_shape=[
          jax.ShapeDtypeStruct((_NCORES, _N, _D2), jnp.float32),
          jax.ShapeDtypeStruct((1, _N), jnp.float32),
          jax.ShapeDtypeStruct((1, _N), jnp.float32),
          jax.ShapeDtypeStruct((1, 2), jnp.float32),
      ],
  )(partial1, b1, w2, att_s, att_d)


def _final(partial2, b2):
  def body(p_ref, b_ref, o_ref):
    ch = _C // 2
    num = jnp.concatenate(
        [p_ref[0, :_N, :ch], p_ref[1, :_N, :ch]], axis=1)
    den = p_ref[0, :_N, ch:ch + 1] + 1e-16
    o = num / den + b_ref[...]
    mx = jnp.max(o, axis=1, keepdims=True)
    shifted = o - mx
    lse = jnp.log(jnp.sum(jnp.exp(shifted), axis=1, keepdims=True))
    o_ref[...] = shifted - lse

  return pl.pallas_call(
      body,
      out_shape=jax.ShapeDtypeStruct((_N, _C), jnp.float32),
  )(partial2, b2)


def _sc_layer(tabs, src, dst, a_s, a_d, m16, d):
  nd = d // 16
  mesh = plsc.VectorSubcoreMesh(core_axis_name="c", subcore_axis_name="s")
  cp = pltpu.CompilerParams()
  if "needs_layout_passes" in pltpu.CompilerParams.__dataclass_fields__:
    cp = dataclasses.replace(cp, needs_layout_passes=False)
  if "use_tc_tiling_on_sc" in pltpu.CompilerParams.__dataclass_fields__:
    cp = dataclasses.replace(cp, use_tc_tiling_on_sc=False)

  @functools.partial(
      pl.kernel,
      mesh=mesh,
      compiler_params=cp,
      out_type=jax.ShapeDtypeStruct((_NCORES, _N, d), jnp.float32),
      scratch_types=[
          pltpu.VMEM((_N,), jnp.float32),
          pltpu.VMEM((_N,), jnp.float32),
          pltpu.VMEM((16,), jnp.float32),
          pltpu.VMEM((_CH,), jnp.int32),
          pltpu.VMEM((_NB, _B), jnp.int32),
          pltpu.VMEM((_DT,), jnp.int32),
          pltpu.VMEM((_B,), jnp.float32),
          pltpu.VMEM((_B, d), jnp.float32),
          pltpu.VMEM((_B, d), jnp.float32),
          pltpu.VMEM_SHARED((_N, d), jnp.float32),
          pltpu.SemaphoreType.DMA,
          pltpu.SemaphoreType.DMA,
          pltpu.SemaphoreType.DMA,
          pltpu.SemaphoreType.DMA,
      ],
  )
  def k(tab_hbm, src_hbm, dst_hbm, as_hbm, ad_hbm, m_hbm, out_hbm,
        as_v, ad_v, m_v, si_v, dr_v, dt_v, w_v, rows_a, rows_b, acc_sh,
        gsa, gsb, ssa, ssb):
    c = lax.axis_index("c")
    s = lax.axis_index("s")

    pltpu.sync_copy(as_hbm, as_v)
    pltpu.sync_copy(ad_hbm, ad_v)
    pltpu.sync_copy(m_hbm, m_v)
    pltpu.sync_copy(src_hbm.at[pl.ds(s * _CH, _CH)], si_v)

    for seg in range(_SEG):
      pltpu.sync_copy(dst_hbm.at[pl.ds(s * _CH + seg * _DT, _DT)], dt_v)

      @pl.loop(0, _DT // _B)
      def _(b):
        for jj in range(_B // 16):
          dr_v[seg * (_DT // _B) + b, pl.ds(jj * 16, 16)] = (
              dt_v[pl.ds(b * _B + jj * 16, 16)])

    @pl.loop(0, _B)
    def _(r):
      for ch in range(nd):
        rows_a[r, pl.ds(ch * 16, 16)] = jnp.zeros((16,), jnp.float32)

    @pl.loop(0, (_NCH + _NSUB - 1) // _NSUB)
    def _(t):
      idx = t * _NSUB + s

      @pl.when(idx < _NCH)
      def _():
        pltpu.sync_copy(rows_a, acc_sh.at[pl.ds(idx * _B, _B)])

    plsc.subcore_barrier()

    mvec = m_v[...]

    def compute_w(blk):
      @pl.loop(0, _B, step=16)
      def _(j):
        sidx = si_v[pl.ds(blk * _B + j, 16)]
        didx = dr_v[blk, pl.ds(j, 16)]
        e = plsc.load_gather(as_v, [sidx]) + plsc.load_gather(ad_v, [didx])
        e = jnp.maximum(e, 0.2 * e)
        w_v[pl.ds(j, 16)] = jnp.exp(e - mvec)

    def scale(rows_v):
      @pl.loop(0, _B)
      def _(i):
        ws = plsc.load_gather(w_v, [jnp.full((16,), 0, jnp.int32) + i])
        for ch in range(nd):
          sl = pl.ds(ch * 16, 16)
          rows_v[i, sl] = rows_v[i, sl] * ws

    @pl.loop(0, _NB, step=2)
    def _(blk):
      @pl.when(blk > 0)
      def _():
        pltpu.make_async_copy(rows_a, acc_sh.at[dr_v.at[0]], ssa).wait()
        pltpu.make_async_copy(rows_b, acc_sh.at[dr_v.at[0]], ssb).wait()

      ga = pltpu.async_copy(
          tab_hbm.at[c].at[si_v.at[pl.ds(blk * _B, _B)]], rows_a, gsa)
      gb = pltpu.async_copy(
          tab_hbm.at[c].at[si_v.at[pl.ds((blk + 1) * _B, _B)]], rows_b, gsb)

      compute_w(blk)
      ga.wait()
      scale(rows_a)
      pltpu.async_copy(rows_a, acc_sh.at[dr_v.at[blk]], ssa, add=True)

      compute_w(blk + 1)
      gb.wait()
      scale(rows_b)
      pltpu.async_copy(rows_b, acc_sh.at[dr_v.at[blk + 1]], ssb, add=True)

    pltpu.make_async_copy(rows_a, acc_sh.at[dr_v.at[0]], ssa).wait()
    pltpu.make_async_copy(rows_b, acc_sh.at[dr_v.at[0]], ssb).wait()

    plsc.subcore_barrier()

    @pl.loop(0, (_NCH + _NSUB - 1) // _NSUB)
    def _(t):
      idx = t * _NSUB + s

      @pl.when(idx < _NCH)
      def _():
        r0 = idx * _B
        pltpu.sync_copy(acc_sh.at[pl.ds(r0, _B)], out_hbm.at[c, pl.ds(r0, _B)])

  return k(tabs, src, dst, a_s, a_d, m16)


def kernel(features, edges, W1, att_src1, att_dst1, b1,
           W2, att_src2, att_dst2, b2):
  src = edges[0]
  dst = edges[1]
  tabs1, a1s, a1d, m1 = _dense1(features, W1, att_src1, att_dst1)
  big_m1 = jnp.maximum(m1[0, 0] + m1[0, 1], 0.0)
  m16_1 = jnp.full((16,), big_m1, jnp.float32)
  part1 = _sc_layer(tabs1, src, dst, a1s.reshape(-1), a1d.reshape(-1),
                    m16_1, _D1)

  tabs2, a2s, a2d, m2 = _mid(part1, b1.reshape(1, _H), W2, att_src2, att_dst2)
  big_m2 = jnp.maximum(m2[0, 0] + m2[0, 1], 0.0)
  m16_2 = jnp.full((16,), big_m2, jnp.float32)
  part2 = _sc_layer(tabs2, src, dst, a2s.reshape(-1), a2d.reshape(-1),
                    m16_2, _D2)

  return _final(part2, b2.reshape(1, _C))


# speedup vs baseline: 37.6550x; 1.3622x over previous
import dataclasses
import functools

import jax
import jax.numpy as jnp
from jax import lax
from jax.experimental import pallas as pl
from jax.experimental.pallas import tpu as pltpu
from jax.experimental.pallas import tpu_sc as plsc

_N = 10000
_E = 320000
_F = 128
_H = 128
_C = 16

_D1 = 80
_D2 = 16

_NCORES = 2
_NSUB = 16
_CH = _E // _NSUB
_B = 80
_NB = _CH // _B
_NCH = _N // _B
_DT = 4000
_SEG = _CH // _DT


def _dense1(features, w1, att_s, att_d):
  def body(x_ref, w_ref, s_ref, d_ref, tab_ref, as_ref, ad_ref, m_ref):
    xw = jnp.dot(x_ref[...], w_ref[...], preferred_element_type=jnp.float32)
    hh = _H // 2
    ones_col = (lax.broadcasted_iota(jnp.int32, (_N, _D1 - hh), 1) == 0)
    ones_col = ones_col.astype(jnp.float32)
    tab_ref[0, :, :hh] = xw[:, :hh]
    tab_ref[0, :, hh:] = ones_col
    tab_ref[1, :, :hh] = xw[:, hh:]
    tab_ref[1, :, hh:] = ones_col
    a_s = jnp.sum(xw * s_ref[...], axis=1)
    a_d = jnp.sum(xw * d_ref[...], axis=1)
    as_ref[...] = a_s[None, :]
    ad_ref[...] = a_d[None, :]
    m_ref[...] = jnp.concatenate(
        [jnp.max(a_s)[None, None], jnp.max(a_d)[None, None]], axis=1)

  return pl.pallas_call(
      body,
      out_shape=[
          jax.ShapeDtypeStruct((_NCORES, _N, _D1), jnp.float32),
          jax.ShapeDtypeStruct((1, _N), jnp.float32),
          jax.ShapeDtypeStruct((1, _N), jnp.float32),
          jax.ShapeDtypeStruct((1, 2), jnp.float32),
      ],
  )(features, w1, att_s, att_d)


def _mid(partial1, b1, w2, att_s, att_d):
  def body(p_ref, b_ref, w_ref, s_ref, d_ref, tab_ref, as_ref, ad_ref, m_ref):
    hh = _H // 2
    num = jnp.concatenate(
        [p_ref[0, :_N, :hh], p_ref[1, :_N, :hh]], axis=1)
    den = p_ref[0, :_N, hh:hh + 1] + 1e-16
    h = jax.nn.relu(num / den + b_ref[...])
    xw = jnp.dot(h, w_ref[...], preferred_element_type=jnp.float32)
    ch = _C // 2
    ones_col = (lax.broadcasted_iota(jnp.int32, (_N, _D2 - ch), 1) == 0)
    ones_col = ones_col.astype(jnp.float32)
    tab_ref[0, :, :ch] = xw[:, :ch]
    tab_ref[0, :, ch:] = ones_col
    tab_ref[1, :, :ch] = xw[:, ch:]
    tab_ref[1, :, ch:] = ones_col
    a_s = jnp.sum(xw * s_ref[...], axis=1)
    a_d = jnp.sum(xw * d_ref[...], axis=1)
    as_ref[...] = a_s[None, :]
    ad_ref[...] = a_d[None, :]
    m_ref[...] = jnp.concatenate(
        [jnp.max(a_s)[None, None], jnp.max(a_d)[None, None]], axis=1)

  return pl.pallas_call(
      body,
      out_shape=[
          jax.ShapeDtypeStruct((_NCORES, _N, _D2), jnp.float32),
          jax.ShapeDtypeStruct((1, _N), jnp.float32),
          jax.ShapeDtypeStruct((1, _N), jnp.float32),
          jax.ShapeDtypeStruct((1, 2), jnp.float32),
      ],
  )(partial1, b1, w2, att_s, att_d)


def _final(partial2, b2):
  def body(p_ref, b_ref, o_ref):
    ch = _C // 2
    num = jnp.concatenate(
        [p_ref[0, :_N, :ch], p_ref[1, :_N, :ch]], axis=1)
    den = p_ref[0, :_N, ch:ch + 1] + 1e-16
    o = num / den + b_ref[...]
    mx = jnp.max(o, axis=1, keepdims=True)
    shifted = o - mx
    lse = jnp.log(jnp.sum(jnp.exp(shifted), axis=1, keepdims=True))
    o_ref[...] = shifted - lse

  return pl.pallas_call(
      body,
      out_shape=jax.ShapeDtypeStruct((_N, _C), jnp.float32),
  )(partial2, b2)


def _sc_layer(tabs, src, dst, a_s, a_d, m16, d):
  nd = d // 16
  mesh = plsc.VectorSubcoreMesh(core_axis_name="c", subcore_axis_name="s")
  cp = pltpu.CompilerParams()
  if "needs_layout_passes" in pltpu.CompilerParams.__dataclass_fields__:
    cp = dataclasses.replace(cp, needs_layout_passes=False)
  if "use_tc_tiling_on_sc" in pltpu.CompilerParams.__dataclass_fields__:
    cp = dataclasses.replace(cp, use_tc_tiling_on_sc=False)

  @functools.partial(
      pl.kernel,
      mesh=mesh,
      compiler_params=cp,
      out_type=jax.ShapeDtypeStruct((_NCORES, _N, d), jnp.float32),
      scratch_types=[
          pltpu.VMEM((_N,), jnp.float32),
          pltpu.VMEM((_N,), jnp.float32),
          pltpu.VMEM((16,), jnp.float32),
          pltpu.VMEM((_CH,), jnp.int32),
          pltpu.VMEM((_NB, _B), jnp.int32),
          pltpu.VMEM((_DT,), jnp.int32),
          pltpu.VMEM((_B,), jnp.float32),
          pltpu.VMEM((_B, d), jnp.float32),
          pltpu.VMEM((_B, d), jnp.float32),
          pltpu.VMEM_SHARED((_N, d), jnp.float32),
          pltpu.SemaphoreType.DMA,
          pltpu.SemaphoreType.DMA,
          pltpu.SemaphoreType.DMA,
          pltpu.SemaphoreType.DMA,
      ],
  )
  def k(tab_hbm, src_hbm, dst_hbm, as_hbm, ad_hbm, m_hbm, out_hbm,
        as_v, ad_v, m_v, si_v, dr_v, dt_v, w_v, rows_a, rows_b, acc_sh,
        gsa, gsb, ssa, ssb):
    c = lax.axis_index("c")
    s = lax.axis_index("s")

    pltpu.sync_copy(as_hbm, as_v)
    pltpu.sync_copy(ad_hbm, ad_v)
    pltpu.sync_copy(m_hbm, m_v)
    pltpu.sync_copy(src_hbm.at[pl.ds(s * _CH, _CH)], si_v)

    for seg in range(_SEG):
      pltpu.sync_copy(dst_hbm.at[pl.ds(s * _CH + seg * _DT, _DT)], dt_v)

      @pl.loop(0, _DT // _B)
      def _(b):
        for jj in range(_B // 16):
          dr_v[seg * (_DT // _B) + b, pl.ds(jj * 16, 16)] = (
              dt_v[pl.ds(b * _B + jj * 16, 16)])

    @pl.loop(0, _B)
    def _(r):
      for ch in range(nd):
        rows_a[r, pl.ds(ch * 16, 16)] = jnp.zeros((16,), jnp.float32)

    @pl.loop(0, (_NCH + _NSUB - 1) // _NSUB)
    def _(t):
      idx = t * _NSUB + s

      @pl.when(idx < _NCH)
      def _():
        pltpu.sync_copy(rows_a, acc_sh.at[pl.ds(idx * _B, _B)])

    plsc.subcore_barrier()

    mvec = m_v[...]

    def compute_w(blk):
      @plsc.parallel_loop(0, _B, 16, unroll=_B // 16)
      def _(j):
        sidx = si_v[pl.ds(blk * _B + j, 16)]
        didx = dr_v[blk, pl.ds(j, 16)]
        e = plsc.load_gather(as_v, [sidx]) + plsc.load_gather(ad_v, [didx])
        e = jnp.maximum(e, 0.2 * e)
        w_v[pl.ds(j, 16)] = jnp.exp(e - mvec)

    def scale(rows_v):
      @plsc.parallel_loop(0, _B, unroll=8)
      def _(i):
        ws = plsc.load_gather(w_v, [jnp.full((16,), 0, jnp.int32) + i])
        for ch in range(nd):
          sl = pl.ds(ch * 16, 16)
          rows_v[i, sl] = rows_v[i, sl] * ws

    @pl.loop(0, _NB, step=2)
    def _(blk):
      @pl.when(blk > 0)
      def _():
        pltpu.make_async_copy(rows_a, acc_sh.at[dr_v.at[0]], ssa).wait()
        pltpu.make_async_copy(rows_b, acc_sh.at[dr_v.at[0]], ssb).wait()

      ga = pltpu.async_copy(
          tab_hbm.at[c].at[si_v.at[pl.ds(blk * _B, _B)]], rows_a, gsa)
      gb = pltpu.async_copy(
          tab_hbm.at[c].at[si_v.at[pl.ds((blk + 1) * _B, _B)]], rows_b, gsb)

      compute_w(blk)
      ga.wait()
      scale(rows_a)
      pltpu.async_copy(rows_a, acc_sh.at[dr_v.at[blk]], ssa, add=True)

      compute_w(blk + 1)
      gb.wait()
      scale(rows_b)
      pltpu.async_copy(rows_b, acc_sh.at[dr_v.at[blk + 1]], ssb, add=True)

    pltpu.make_async_copy(rows_a, acc_sh.at[dr_v.at[0]], ssa).wait()
    pltpu.make_async_copy(rows_b, acc_sh.at[dr_v.at[0]], ssb).wait()

    plsc.subcore_barrier()

    @pl.loop(0, (_NCH + _NSUB - 1) // _NSUB)
    def _(t):
      idx = t * _NSUB + s

      @pl.when(idx < _NCH)
      def _():
        r0 = idx * _B
        pltpu.sync_copy(acc_sh.at[pl.ds(r0, _B)], out_hbm.at[c, pl.ds(r0, _B)])

  return k(tabs, src, dst, a_s, a_d, m16)


def kernel(features, edges, W1, att_src1, att_dst1, b1,
           W2, att_src2, att_dst2, b2):
  src = edges[0]
  dst = edges[1]
  tabs1, a1s, a1d, m1 = _dense1(features, W1, att_src1, att_dst1)
  big_m1 = jnp.maximum(m1[0, 0] + m1[0, 1], 0.0)
  m16_1 = jnp.full((16,), big_m1, jnp.float32)
  part1 = _sc_layer(tabs1, src, dst, a1s.reshape(-1), a1d.reshape(-1),
                    m16_1, _D1)

  tabs2, a2s, a2d, m2 = _mid(part1, b1.reshape(1, _H), W2, att_src2, att_dst2)
  big_m2 = jnp.maximum(m2[0, 0] + m2[0, 1], 0.0)
  m16_2 = jnp.full((16,), big_m2, jnp.float32)
  part2 = _sc_layer(tabs2, src, dst, a2s.reshape(-1), a2d.reshape(-1),
                    m16_2, _D2)

  return _final(part2, b2.reshape(1, _C))


# speedup vs baseline: 41.8025x; 1.1101x over previous
import dataclasses
import functools

import jax
import jax.numpy as jnp
from jax import lax
from jax.experimental import pallas as pl
from jax.experimental.pallas import tpu as pltpu
from jax.experimental.pallas import tpu_sc as plsc

_N = 10000
_E = 320000
_F = 128
_H = 128
_C = 16

_D1 = 80
_D2 = 16

_NCORES = 2
_NSUB = 16
_CH = _E // _NSUB
_B = 80
_NB = _CH // _B
_NCH = _N // _B
_DT = 4000
_SEG = _CH // _DT


def _dense1(features, w1, att_s, att_d):
  def body(x_ref, w_ref, s_ref, d_ref, tab_ref, as_ref, ad_ref, m_ref):
    xw = jnp.dot(x_ref[...], w_ref[...], preferred_element_type=jnp.float32)
    hh = _H // 2
    ones_col = (lax.broadcasted_iota(jnp.int32, (_N, _D1 - hh), 1) == 0)
    ones_col = ones_col.astype(jnp.float32)
    tab_ref[0, :, :hh] = xw[:, :hh]
    tab_ref[0, :, hh:] = ones_col
    tab_ref[1, :, :hh] = xw[:, hh:]
    tab_ref[1, :, hh:] = ones_col
    a_s = jnp.sum(xw * s_ref[...], axis=1)
    a_d = jnp.sum(xw * d_ref[...], axis=1)
    as_ref[...] = a_s[None, :]
    ad_ref[...] = a_d[None, :]
    m_ref[...] = jnp.concatenate(
        [jnp.max(a_s)[None, None], jnp.max(a_d)[None, None]], axis=1)

  return pl.pallas_call(
      body,
      out_shape=[
          jax.ShapeDtypeStruct((_NCORES, _N, _D1), jnp.float32),
          jax.ShapeDtypeStruct((1, _N), jnp.float32),
          jax.ShapeDtypeStruct((1, _N), jnp.float32),
          jax.ShapeDtypeStruct((1, 2), jnp.float32),
      ],
  )(features, w1, att_s, att_d)


def _mid(partial1, b1, w2, att_s, att_d):
  def body(p_ref, b_ref, w_ref, s_ref, d_ref, tab_ref, as_ref, ad_ref, m_ref):
    hh = _H // 2
    num = jnp.concatenate(
        [p_ref[0, :_N, :hh], p_ref[1, :_N, :hh]], axis=1)
    den = p_ref[0, :_N, hh:hh + 1] + 1e-16
    h = jax.nn.relu(num / den + b_ref[...])
    xw = jnp.dot(h, w_ref[...], preferred_element_type=jnp.float32)
    ch = _C // 2
    ones_col = (lax.broadcasted_iota(jnp.int32, (_N, _D2 - ch), 1) == 0)
    ones_col = ones_col.astype(jnp.float32)
    tab_ref[0, :, :ch] = xw[:, :ch]
    tab_ref[0, :, ch:] = ones_col
    tab_ref[1, :, :ch] = xw[:, ch:]
    tab_ref[1, :, ch:] = ones_col
    a_s = jnp.sum(xw * s_ref[...], axis=1)
    a_d = jnp.sum(xw * d_ref[...], axis=1)
    as_ref[...] = a_s[None, :]
    ad_ref[...] = a_d[None, :]
    m_ref[...] = jnp.concatenate(
        [jnp.max(a_s)[None, None], jnp.max(a_d)[None, None]], axis=1)

  return pl.pallas_call(
      body,
      out_shape=[
          jax.ShapeDtypeStruct((_NCORES, _N, _D2), jnp.float32),
          jax.ShapeDtypeStruct((1, _N), jnp.float32),
          jax.ShapeDtypeStruct((1, _N), jnp.float32),
          jax.ShapeDtypeStruct((1, 2), jnp.float32),
      ],
  )(partial1, b1, w2, att_s, att_d)


def _final(partial2, b2):
  def body(p_ref, b_ref, o_ref):
    ch = _C // 2
    num = jnp.concatenate(
        [p_ref[0, :_N, :ch], p_ref[1, :_N, :ch]], axis=1)
    den = p_ref[0, :_N, ch:ch + 1] + 1e-16
    o = num / den + b_ref[...]
    mx = jnp.max(o, axis=1, keepdims=True)
    shifted = o - mx
    lse = jnp.log(jnp.sum(jnp.exp(shifted), axis=1, keepdims=True))
    o_ref[...] = shifted - lse

  return pl.pallas_call(
      body,
      out_shape=jax.ShapeDtypeStruct((_N, _C), jnp.float32),
  )(partial2, b2)


def _sc_layer(tabs, src, dst, a_s, a_d, m16, d, nbuf):
  nd = d // 16
  assert _NB % nbuf == 0
  mesh = plsc.VectorSubcoreMesh(core_axis_name="c", subcore_axis_name="s")
  cp = pltpu.CompilerParams()
  if "needs_layout_passes" in pltpu.CompilerParams.__dataclass_fields__:
    cp = dataclasses.replace(cp, needs_layout_passes=False)
  if "use_tc_tiling_on_sc" in pltpu.CompilerParams.__dataclass_fields__:
    cp = dataclasses.replace(cp, use_tc_tiling_on_sc=False)

  @functools.partial(
      pl.kernel,
      mesh=mesh,
      compiler_params=cp,
      out_type=jax.ShapeDtypeStruct((_NCORES, _N, d), jnp.float32),
      scratch_types=[
          pltpu.VMEM((_N,), jnp.float32),
          pltpu.VMEM((_N,), jnp.float32),
          pltpu.VMEM((16,), jnp.float32),
          pltpu.VMEM((_CH,), jnp.int32),
          pltpu.VMEM((_NB, _B), jnp.int32),
          pltpu.VMEM((_DT,), jnp.int32),
          pltpu.VMEM((_B,), jnp.float32),
      ] + [pltpu.VMEM((_B, d), jnp.float32) for _ in range(nbuf)]
      + [pltpu.VMEM_SHARED((_N, d), jnp.float32)]
      + [pltpu.SemaphoreType.DMA for _ in range(2 * nbuf)],
  )
  def k(tab_hbm, src_hbm, dst_hbm, as_hbm, ad_hbm, m_hbm, out_hbm,
        as_v, ad_v, m_v, si_v, dr_v, dt_v, w_v, *rest):
    bufs = rest[:nbuf]
    acc_sh = rest[nbuf]
    gsems = rest[nbuf + 1:2 * nbuf + 1]
    ssems = rest[2 * nbuf + 1:]
    rows_a = bufs[0]
    c = lax.axis_index("c")
    s = lax.axis_index("s")

    pltpu.sync_copy(as_hbm, as_v)
    pltpu.sync_copy(ad_hbm, ad_v)
    pltpu.sync_copy(m_hbm, m_v)
    pltpu.sync_copy(src_hbm.at[pl.ds(s * _CH, _CH)], si_v)

    for seg in range(_SEG):
      pltpu.sync_copy(dst_hbm.at[pl.ds(s * _CH + seg * _DT, _DT)], dt_v)

      @pl.loop(0, _DT // _B)
      def _(b):
        for jj in range(_B // 16):
          dr_v[seg * (_DT // _B) + b, pl.ds(jj * 16, 16)] = (
              dt_v[pl.ds(b * _B + jj * 16, 16)])

    @pl.loop(0, _B)
    def _(r):
      for ch in range(nd):
        rows_a[r, pl.ds(ch * 16, 16)] = jnp.zeros((16,), jnp.float32)

    @pl.loop(0, (_NCH + _NSUB - 1) // _NSUB)
    def _(t):
      idx = t * _NSUB + s

      @pl.when(idx < _NCH)
      def _():
        pltpu.sync_copy(rows_a, acc_sh.at[pl.ds(idx * _B, _B)])

    plsc.subcore_barrier()

    mvec = m_v[...]

    def compute_w(blk):
      @plsc.parallel_loop(0, _B, 16, unroll=_B // 16)
      def _(j):
        sidx = si_v[pl.ds(blk * _B + j, 16)]
        didx = dr_v[blk, pl.ds(j, 16)]
        e = plsc.load_gather(as_v, [sidx]) + plsc.load_gather(ad_v, [didx])
        e = jnp.maximum(e, 0.2 * e)
        w_v[pl.ds(j, 16)] = jnp.exp(e - mvec)

    def scale(rows_v):
      @plsc.parallel_loop(0, _B, unroll=8)
      def _(i):
        ws = plsc.load_gather(w_v, [jnp.full((16,), 0, jnp.int32) + i])
        for ch in range(nd):
          sl = pl.ds(ch * 16, 16)
          rows_v[i, sl] = rows_v[i, sl] * ws

    @pl.loop(0, _NB, step=nbuf)
    def _(blk):
      @pl.when(blk > 0)
      def _():
        for q in range(nbuf):
          pltpu.make_async_copy(bufs[q], acc_sh.at[dr_v.at[0]],
                                ssems[q]).wait()

      gs = []
      for q in range(nbuf):
        gs.append(pltpu.async_copy(
            tab_hbm.at[c].at[si_v.at[pl.ds((blk + q) * _B, _B)]],
            bufs[q], gsems[q]))

      for q in range(nbuf):
        compute_w(blk + q)
        gs[q].wait()
        scale(bufs[q])
        pltpu.async_copy(bufs[q], acc_sh.at[dr_v.at[blk + q]],
                         ssems[q], add=True)

    for q in range(nbuf):
      pltpu.make_async_copy(bufs[q], acc_sh.at[dr_v.at[0]], ssems[q]).wait()

    plsc.subcore_barrier()

    @pl.loop(0, (_NCH + _NSUB - 1) // _NSUB)
    def _(t):
      idx = t * _NSUB + s

      @pl.when(idx < _NCH)
      def _():
        r0 = idx * _B
        pltpu.sync_copy(acc_sh.at[pl.ds(r0, _B)], out_hbm.at[c, pl.ds(r0, _B)])

  return k(tabs, src, dst, a_s, a_d, m16)


def kernel(features, edges, W1, att_src1, att_dst1, b1,
           W2, att_src2, att_dst2, b2):
  src = edges[0]
  dst = edges[1]
  tabs1, a1s, a1d, m1 = _dense1(features, W1, att_src1, att_dst1)
  big_m1 = jnp.maximum(m1[0, 0] + m1[0, 1], 0.0)
  m16_1 = jnp.full((16,), big_m1, jnp.float32)
  part1 = _sc_layer(tabs1, src, dst, a1s.reshape(-1), a1d.reshape(-1),
                    m16_1, _D1, 2)

  tabs2, a2s, a2d, m2 = _mid(part1, b1.reshape(1, _H), W2, att_src2, att_dst2)
  big_m2 = jnp.maximum(m2[0, 0] + m2[0, 1], 0.0)
  m16_2 = jnp.full((16,), big_m2, jnp.float32)
  part2 = _sc_layer(tabs2, src, dst, a2s.reshape(-1), a2d.reshape(-1),
                    m16_2, _D2, 5)

  return _final(part2, b2.reshape(1, _C))
